# baseline (speedup 1.0000x reference)
import functools

import jax
import jax.numpy as jnp
from jax import lax
from jax.experimental import pallas as pl
from jax.experimental.pallas import tpu as pltpu
from jax.experimental.pallas import tpu_sc as plsc

_D = 64
_LANES = 128
_ROWS_PER_CHUNK = 8
_CHUNK = _ROWS_PER_CHUNK * _LANES

_B = 4096
_S = 200
_N_IDX = _B * _S
_N_ROWS = _N_IDX // _LANES
_NC = 2
_NS = 16
_NW = _NC * _NS
_ROWS_PER_W = _N_ROWS // _NW
_N_CHUNKS = _ROWS_PER_W // _ROWS_PER_CHUNK

_mesh = plsc.VectorSubcoreMesh(core_axis_name="c", subcore_axis_name="s")


@functools.partial(
    pl.kernel,
    mesh=_mesh,
    out_type=jax.ShapeDtypeStruct((_N_IDX, _D), jnp.float32),
    scratch_types=[
        pltpu.VMEM((_ROWS_PER_CHUNK, _LANES), jnp.int32),
        pltpu.VMEM((_CHUNK, _D), jnp.float32),
        pltpu.SemaphoreType.DMA,
    ],
    compiler_params=pltpu.CompilerParams(use_tc_tiling_on_sc=False),
)
def _sc_gather(idx_hbm, table_hbm, out_hbm, idx_v, rows_v, sem):
    wid = lax.axis_index("s") * _NC + lax.axis_index("c")
    row0 = wid * _ROWS_PER_W

    def chunk_body(i, carry):
        r = row0 + i * _ROWS_PER_CHUNK
        pltpu.sync_copy(idx_hbm.at[pl.ds(r, _ROWS_PER_CHUNK)], idx_v)
        copies = [
            pltpu.async_copy(
                table_hbm.at[idx_v.at[j]],
                rows_v.at[pl.ds(j * _LANES, _LANES)],
                sem,
            )
            for j in range(_ROWS_PER_CHUNK)
        ]
        for c in copies:
            c.wait()
        pltpu.sync_copy(rows_v, out_hbm.at[pl.ds(r * _LANES, _CHUNK)])
        return carry

    lax.fori_loop(0, _N_CHUNKS, chunk_body, 0)


def _unpack_t(x_ref, o_ref):
    x = x_ref[...]
    o_ref[...] = (
        x.reshape(8, 64, 2, 64).transpose(0, 3, 1, 2).reshape(8, 64, 128)
    )


def kernel(sentence, elmo_tensor, word_embed):
    del elmo_tensor
    st = sentence.T
    idx = (
        st.reshape(_S // 8, 8, _B // _LANES, _LANES)
        .transpose(0, 2, 1, 3)
        .reshape(_N_ROWS, _LANES)
    )
    out_lin = _sc_gather(idx, word_embed)
    pairs = out_lin.reshape(_N_IDX // 2, 2 * _D)
    ot = pl.pallas_call(
        _unpack_t,
        grid=(_S // 8, _B // _LANES),
        in_specs=[pl.BlockSpec((512, 128), lambda a, t: (a * 32 + t, 0))],
        out_specs=pl.BlockSpec((8, _D, _LANES), lambda a, t: (a, 0, t)),
        out_shape=jax.ShapeDtypeStruct((_S, _D, _B), jnp.float32),
    )(pairs)
    return ot.transpose(2, 0, 1)


# speedup vs baseline: 6.3148x; 6.3148x over previous
import functools

import jax
import jax.numpy as jnp
from jax import lax
from jax.experimental import pallas as pl
from jax.experimental.pallas import tpu as pltpu
from jax.experimental.pallas import tpu_sc as plsc

_D = 64
_LANES = 128
_ROWS_PER_CHUNK = 8
_CHUNK = _ROWS_PER_CHUNK * _LANES

_B = 4096
_S = 200
_N_IDX = _B * _S
_N_ROWS = _N_IDX // _LANES
_N_PK = _N_IDX // 2
_NC = 2
_NS = 16
_NW = _NC * _NS
_ROWS_PER_W = _N_ROWS // _NW
_N_CHUNKS = _ROWS_PER_W // _ROWS_PER_CHUNK

_mesh = plsc.VectorSubcoreMesh(core_axis_name="c", subcore_axis_name="s")


@functools.partial(
    pl.kernel,
    mesh=_mesh,
    out_type=jax.ShapeDtypeStruct((_N_PK, 2 * _D), jnp.float32),
    scratch_types=[
        pltpu.VMEM((_ROWS_PER_CHUNK, _LANES), jnp.int32),
        pltpu.VMEM((_CHUNK, _D), jnp.float32),
        pltpu.SemaphoreType.DMA,
        pltpu.SemaphoreType.DMA,
    ],
    compiler_params=pltpu.CompilerParams(use_tc_tiling_on_sc=False),
)
def _sc_gather(idx_hbm, table_hbm, out_hbm, idx_v, rows_v, sem, sem_w):
    wid = lax.axis_index("s") * _NC + lax.axis_index("c")
    row0 = wid * _ROWS_PER_W

    def chunk_body(i, carry):
        r = row0 + i * _ROWS_PER_CHUNK
        pltpu.sync_copy(idx_hbm.at[pl.ds(r, _ROWS_PER_CHUNK)], idx_v)
        copies = [
            pltpu.async_copy(
                table_hbm.at[idx_v.at[j]],
                rows_v.at[pl.ds(j * _LANES, _LANES)],
                sem,
            )
            for j in range(_ROWS_PER_CHUNK)
        ]
        p0 = r * _D
        writes = []
        for j in range(_ROWS_PER_CHUNK):
            copies[j].wait()
            writes.append(
                pltpu.async_copy(
                    rows_v.at[pl.ds(j * _LANES, _LANES)],
                    out_hbm.at[
                        pl.ds(p0 + (j // 2) * _LANES, _LANES),
                        pl.ds((j % 2) * _D, _D),
                    ],
                    sem_w,
                )
            )
        for w in writes:
            w.wait()
        return carry

    lax.fori_loop(0, _N_CHUNKS, chunk_body, 0)


def _unpack_t(x_ref, o_ref):
    x = x_ref[...]
    o_ref[...] = (
        x.T.reshape(_LANES, 4, _LANES).transpose(1, 0, 2).reshape(8, _D, _LANES)
    )


def kernel(sentence, elmo_tensor, word_embed):
    del elmo_tensor
    st = sentence.T
    idx = (
        st.reshape(_S // 8, 8, _B // _LANES, _LANES)
        .transpose(0, 2, 1, 3)
        .reshape(_N_ROWS, _LANES)
    )
    pairs = _sc_gather(idx, word_embed)
    ot = pl.pallas_call(
        _unpack_t,
        grid=(_S // 8, _B // _LANES),
        in_specs=[pl.BlockSpec((512, 128), lambda a, t: (a * 32 + t, 0))],
        out_specs=pl.BlockSpec((8, _D, _LANES), lambda a, t: (a, 0, t)),
        out_shape=jax.ShapeDtypeStruct((_S, _D, _B), jnp.float32),
    )(pairs)
    return ot.transpose(2, 0, 1)


# speedup vs baseline: 6.5714x; 1.0406x over previous
import functools

import jax
import jax.numpy as jnp
from jax import lax
from jax.experimental import pallas as pl
from jax.experimental.pallas import tpu as pltpu
from jax.experimental.pallas import tpu_sc as plsc

_D = 64
_LANES = 128
_ROWS_PER_CHUNK = 8
_CHUNK = _ROWS_PER_CHUNK * _LANES

_B = 4096
_S = 200
_N_IDX = _B * _S
_N_ROWS = _N_IDX // _LANES
_NC = 2
_NS = 16
_NW = _NC * _NS
_ROWS_PER_W = _N_ROWS // _NW
_N_CHUNKS = _ROWS_PER_W // _ROWS_PER_CHUNK

_mesh = plsc.VectorSubcoreMesh(core_axis_name="c", subcore_axis_name="s")


@functools.partial(
    pl.kernel,
    mesh=_mesh,
    out_type=jax.ShapeDtypeStruct((_B, _S, _D), jnp.float32),
    scratch_types=[
        pltpu.VMEM((_ROWS_PER_CHUNK, _LANES), jnp.int32),
        pltpu.VMEM((_CHUNK, _D), jnp.float32),
        pltpu.SemaphoreType.DMA,
        pltpu.SemaphoreType.DMA,
    ],
    compiler_params=pltpu.CompilerParams(use_tc_tiling_on_sc=False),
)
def _sc_gather(idx_hbm, table_hbm, out_hbm, idx_v, rows_v, sem, sem_w):
    wid = lax.axis_index("s") * _NC + lax.axis_index("c")
    row0 = wid * _ROWS_PER_W

    def chunk_body(i, carry):
        r = row0 + i * _ROWS_PER_CHUNK
        pltpu.sync_copy(idx_hbm.at[pl.ds(r, _ROWS_PER_CHUNK)], idx_v)
        copies = [
            pltpu.async_copy(
                table_hbm.at[idx_v.at[j]],
                rows_v.at[pl.ds(j * _LANES, _LANES)],
                sem,
            )
            for j in range(_ROWS_PER_CHUNK)
        ]
        writes = []
        for j in range(_ROWS_PER_CHUNK):
            copies[j].wait()
            q = r + j
            a = q // 256
            t = (q // 8) % 32
            b = q % 8
            writes.append(
                pltpu.async_copy(
                    rows_v.at[pl.ds(j * _LANES, _LANES)],
                    out_hbm.at[pl.ds(t * _LANES, _LANES), 8 * a + b],
                    sem_w,
                )
            )
        for w in writes:
            w.wait()
        return carry

    lax.fori_loop(0, _N_CHUNKS, chunk_body, 0)


def kernel(sentence, elmo_tensor, word_embed):
    del elmo_tensor
    st = sentence.T
    idx = (
        st.reshape(_S // 8, 8, _B // _LANES, _LANES)
        .transpose(0, 2, 1, 3)
        .reshape(_N_ROWS, _LANES)
    )
    return _sc_gather(idx, word_embed)
